# baseline (speedup 1.0000x reference)
import functools

import jax
import jax.numpy as jnp
from jax import lax
from jax.experimental import pallas as pl
from jax.experimental.pallas import tpu as pltpu
from jax.experimental.pallas import tpu_sc as plsc

_VOCAB = 100000
_EMBED = 128
_B = 32

_VBLK = 12800
_NBLK = -(-_VOCAB // _VBLK)

_ROWS_PER_WORKER = 8
_NWORKERS = _B // _ROWS_PER_WORKER


def _make_sc_gather():
    mesh = plsc.VectorSubcoreMesh(core_axis_name="c", subcore_axis_name="s")
    info = plsc.get_sparse_core_info()
    nc = info.num_cores

    @functools.partial(
        pl.kernel,
        mesh=mesh,
        out_type=jax.ShapeDtypeStruct((_B, _EMBED), jnp.float32),
        scratch_types=[
            pltpu.VMEM((_ROWS_PER_WORKER,), jnp.int32),
            pltpu.VMEM((_ROWS_PER_WORKER, _EMBED), jnp.float32),
            pltpu.SemaphoreType.DMA,
        ],
    )
    def gather_k(idx_hbm, table_hbm, out_hbm, idx_v, rows_v, sem):
        wid = lax.axis_index("s") * nc + lax.axis_index("c")

        @pl.when(wid < _NWORKERS)
        def _():
            base = wid * _ROWS_PER_WORKER
            pltpu.sync_copy(idx_hbm.at[pl.ds(base, _ROWS_PER_WORKER)], idx_v)
            pltpu.async_copy(table_hbm.at[idx_v], rows_v, sem).wait()
            pltpu.sync_copy(rows_v, out_hbm.at[pl.ds(base, _ROWS_PER_WORKER)])

    return gather_k


def _proj_body(h_ref, w_ref, b_ref, o_ref):
    res = lax.dot_general(
        h_ref[...],
        w_ref[...],
        dimension_numbers=(((1,), (1,)), ((), ())),
        preferred_element_type=jnp.float32,
    ) + b_ref[...]
    o_ref[...] = res[:, None, :]


def _projection(h, W, b2):
    return pl.pallas_call(
        _proj_body,
        grid=(_NBLK,),
        in_specs=[
            pl.BlockSpec((_B, _EMBED), lambda i: (0, 0)),
            pl.BlockSpec((_VBLK, _EMBED), lambda i: (i, 0)),
            pl.BlockSpec((1, _VBLK), lambda i: (0, i)),
        ],
        out_specs=pl.BlockSpec((_B, 1, _VBLK), lambda i: (0, 0, i)),
        out_shape=jax.ShapeDtypeStruct((_B, 1, _VOCAB), jnp.float32),
    )(h, W, b2)


def kernel(x, embed, W, b):
    idx = x.reshape(_B).astype(jnp.int32)
    h = _make_sc_gather()(idx, embed)
    return _projection(h, W, b.reshape(1, _VOCAB))
